# baseline (speedup 1.0000x reference)
import functools

import jax
import jax.numpy as jnp
from jax import lax
from jax.experimental import pallas as pl
from jax.experimental.pallas import tpu as pltpu
from jax.experimental.pallas import tpu_sc as plsc

EMBED_DIM = 32
NC = 2
NS = 16
NW = NC * NS

ROW = 128
SUB = 8
CHUNK = ROW * SUB
NBUF = 3


def _compact_table(table):
    V = table.shape[0]
    BR = 512

    def body(t_ref, o_ref):
        x = t_ref[...].reshape(BR // 4, 4, 32)
        for q in range(4):
            o_ref[:, 32 * q:32 * (q + 1)] = x[:, q, :]

    return pl.pallas_call(
        body,
        grid=(pl.cdiv(V, BR),),
        in_specs=[pl.BlockSpec((BR, 32), lambda i: (i, 0))],
        out_specs=pl.BlockSpec((BR // 4, 128), lambda i: (i, 0)),
        out_shape=jax.ShapeDtypeStruct((V // 4, 128), jnp.float32),
    )(table)


@functools.partial(jax.jit, static_argnums=(2,))
def _gather(idx2d, table, total):
    per_tile = total // NW
    idx_rows = per_tile // ROW
    n_chunks = per_tile // CHUNK
    mesh = plsc.VectorSubcoreMesh(core_axis_name="c", subcore_axis_name="s")

    @functools.partial(
        pl.kernel,
        mesh=mesh,
        out_type=jax.ShapeDtypeStruct((total, 128), jnp.float32),
        scratch_types=[
            pltpu.VMEM((idx_rows, ROW), jnp.int32),
            [pltpu.VMEM((CHUNK, EMBED_DIM), jnp.float32) for _ in range(NBUF)],
            [pltpu.SemaphoreType.DMA for _ in range(NBUF)],
        ],
        compiler_params=pltpu.CompilerParams(use_tc_tiling_on_sc=False),
    )
    def body(idx_hbm, table_hbm, out_hbm, idx_v, rows, gsem):
        wid = lax.axis_index("s") * NC + lax.axis_index("c")
        base_row = wid * idx_rows

        pltpu.sync_copy(idx_hbm.at[pl.ds(base_row, idx_rows)], idx_v)

        def fire(n, p):
            for j in range(SUB):
                pltpu.async_copy(
                    table_hbm.at[idx_v.at[n * SUB + j]],
                    rows[p].at[pl.ds(j * ROW, ROW)],
                    gsem[p],
                )

        def out_slice(n):
            return out_hbm.at[
                pl.ds((base_row + n * SUB) * ROW, CHUNK), pl.ds(0, EMBED_DIM)
            ]

        def drain_and_writeback(n, p):
            pltpu.make_async_copy(out_slice(n), rows[p], gsem[p]).wait()
            pltpu.sync_copy(rows[p], out_slice(n))

        n_slots = n_chunks + 1
        n_iters = (n_slots + NBUF - 1) // NBUF

        def step(it, carry):
            for b in range(NBUF):
                k = it * NBUF + b

                @pl.when(k < n_chunks)
                def _():
                    fire(k, b)

                @pl.when(jnp.logical_and(k >= 1, k < n_slots))
                def _():
                    drain_and_writeback(k - 1, (b - 1) % NBUF)

            return carry

        lax.fori_loop(0, n_iters, step, 0)

    return body(idx2d, table)


def kernel(input, suffixed, pref, chrs, table):
    B, L = input.shape
    total = B * L
    idx2d = input.reshape(total // ROW, ROW).astype(jnp.int32)
    V = table.shape[0]
    tlin = _compact_table(table).reshape(V, EMBED_DIM)
    wide = _gather(idx2d, tlin, total)
    out = jax.lax.slice(wide, (0, 0), (total, EMBED_DIM))
    return out.reshape(B, L, EMBED_DIM)


# speedup vs baseline: 2.3552x; 2.3552x over previous
import functools

import jax
import jax.numpy as jnp
from jax import lax
from jax.experimental import pallas as pl
from jax.experimental.pallas import tpu as pltpu
from jax.experimental.pallas import tpu_sc as plsc

EMBED_DIM = 32
NC = 2
NS = 16
NW = NC * NS

ROW = 128
SUB = 8
CHUNK = ROW * SUB
NBUF = 3


@functools.partial(jax.jit, static_argnums=(2,))
def _gather(idx2d, table, total):
    per_tile = total // NW
    idx_rows = per_tile // ROW
    n_chunks = per_tile // CHUNK
    mesh = plsc.VectorSubcoreMesh(core_axis_name="c", subcore_axis_name="s")

    @functools.partial(
        pl.kernel,
        mesh=mesh,
        out_type=jax.ShapeDtypeStruct((total, 128), jnp.float32),
        scratch_types=[
            pltpu.VMEM((idx_rows, ROW), jnp.int32),
            [pltpu.VMEM((CHUNK, EMBED_DIM), jnp.float32) for _ in range(NBUF)],
            [pltpu.SemaphoreType.DMA for _ in range(NBUF)],
        ],
        compiler_params=pltpu.CompilerParams(use_tc_tiling_on_sc=False),
    )
    def body(idx_hbm, table_hbm, out_hbm, idx_v, rows, gsem):
        wid = lax.axis_index("s") * NC + lax.axis_index("c")
        base_row = wid * idx_rows

        pltpu.sync_copy(idx_hbm.at[pl.ds(base_row, idx_rows)], idx_v)

        def fire(n, p):
            for j in range(SUB):
                pltpu.async_copy(
                    table_hbm.at[idx_v.at[n * SUB + j]],
                    rows[p].at[pl.ds(j * ROW, ROW)],
                    gsem[p],
                )

        def out_slice(n):
            return out_hbm.at[
                pl.ds((base_row + n * SUB) * ROW, CHUNK), pl.ds(0, EMBED_DIM)
            ]

        def drain_and_writeback(n, p):
            pltpu.make_async_copy(out_slice(n), rows[p], gsem[p]).wait()
            pltpu.sync_copy(rows[p], out_slice(n))

        n_slots = n_chunks + 1
        n_iters = (n_slots + NBUF - 1) // NBUF

        def step(it, carry):
            for b in range(NBUF):
                k = it * NBUF + b

                @pl.when(k < n_chunks)
                def _():
                    fire(k, b)

                @pl.when(jnp.logical_and(k >= 1, k < n_slots))
                def _():
                    drain_and_writeback(k - 1, (b - 1) % NBUF)

            return carry

        lax.fori_loop(0, n_iters, step, 0)

    return body(idx2d, table)


def kernel(input, suffixed, pref, chrs, table):
    B, L = input.shape
    total = B * L
    idx2d = input.reshape(total // ROW, ROW).astype(jnp.int32)
    wide = _gather(idx2d, table, total)
    out = jax.lax.slice(wide, (0, 0), (total, EMBED_DIM))
    return out.reshape(B, L, EMBED_DIM)
